# baseline (speedup 1.0000x reference)
import functools

import jax
import jax.numpy as jnp
from jax.experimental import pallas as pl
from jax.experimental.pallas import tpu as pltpu


def _hgnn_kernel(x_ref, w_ref, b_ref, dv_ref, de_ref, h_ref, out_ref,
                 xn_ref, *, num_blocks, half_m):
    i = pl.program_id(0)

    @pl.when(i == 0)
    def _prologue():
        xm = jnp.dot(x_ref[...].astype(jnp.bfloat16),
                     w_ref[...].astype(jnp.bfloat16),
                     preferred_element_type=jnp.float32) + b_ref[...]
        xn_ref[...] = (xm * dv_ref[...]).astype(jnp.bfloat16)
        out_ref[...] = jnp.zeros_like(out_ref)

    des = jnp.sqrt(de_ref[...])
    xn = xn_ref[...]
    for half in range(2):
        sl = slice(half * half_m, (half + 1) * half_m)
        hb = (h_ref[:, sl] * des[:, sl]).astype(jnp.bfloat16)
        e2 = jax.lax.dot_general(
            hb, xn,
            dimension_numbers=(((0,), (0,)), ((), ())),
            preferred_element_type=jnp.float32)
        out_ref[...] += jnp.dot(hb, e2.astype(jnp.bfloat16),
                                preferred_element_type=jnp.float32)

    @pl.when(i == num_blocks - 1)
    def _epilogue():
        xm = jnp.dot(x_ref[...].astype(jnp.bfloat16),
                     w_ref[...].astype(jnp.bfloat16),
                     preferred_element_type=jnp.float32) + b_ref[...]
        out_ref[...] = out_ref[...] * dv_ref[...] + xm


@jax.jit
def kernel(x, H, dv_inv, de_inv, weight, bias):
    N, d_in = x.shape
    M = H.shape[1]
    d_out = weight.shape[1]

    Mb = 512
    while M % Mb != 0:
        Mb //= 2
    num_blocks = M // Mb
    half_m = Mb // 2

    dv2 = dv_inv.reshape(N, 1)
    de2 = de_inv.reshape(1, M)
    b2 = bias.reshape(1, d_out)

    out = pl.pallas_call(
        functools.partial(_hgnn_kernel, num_blocks=num_blocks, half_m=half_m),
        grid=(num_blocks,),
        in_specs=[
            pl.BlockSpec((N, d_in), lambda i: (0, 0)),
            pl.BlockSpec((d_in, d_out), lambda i: (0, 0)),
            pl.BlockSpec((1, d_out), lambda i: (0, 0)),
            pl.BlockSpec((N, 1), lambda i: (0, 0)),
            pl.BlockSpec((1, Mb), lambda i: (0, i)),
            pl.BlockSpec((N, Mb), lambda i: (0, i)),
        ],
        out_specs=pl.BlockSpec((N, d_out), lambda i: (0, 0)),
        out_shape=jax.ShapeDtypeStruct((N, d_out), jnp.float32),
        scratch_shapes=[
            pltpu.VMEM((N, d_out), jnp.bfloat16),
        ],
        compiler_params=pltpu.CompilerParams(
            dimension_semantics=("arbitrary",),
            vmem_limit_bytes=110 * 1024 * 1024,
        ),
    )(x, weight, b2, dv2, de2, H)
    return out
